# baseline (speedup 1.0000x reference)
import functools

import jax
import jax.numpy as jnp
from jax import lax
from jax.experimental import pallas as pl
from jax.experimental.pallas import tpu as pltpu
from jax.experimental.pallas import tpu_sc as plsc

NC, NS, L = 2, 16, 16
NW = NC * NS

B, S, D = 4, 2048, 768
N = B * S
PER_W = N // NW
CHUNK = 64
NCHUNK = PER_W // CHUNK

_mesh = plsc.VectorSubcoreMesh(core_axis_name="c", subcore_axis_name="s")


@functools.partial(
    pl.kernel,
    out_type=jax.ShapeDtypeStruct((N, D), jnp.float32),
    mesh=_mesh,
    scratch_types=[
        pltpu.VMEM((CHUNK,), jnp.int32),
        pltpu.VMEM((CHUNK,), jnp.int32),
        pltpu.VMEM((CHUNK, D), jnp.float32),
        pltpu.VMEM((CHUNK, D), jnp.float32),
        pltpu.SemaphoreType.DMA,
        pltpu.SemaphoreType.DMA,
    ],
)
def _embed_sum(ids_hbm, pos_hbm, tok_tbl, pos_tbl, out_hbm,
               ids_c, pos_c, tok_rows, pos_rows, sem_t, sem_p):
    wid = lax.axis_index("s") * NC + lax.axis_index("c")
    base = wid * PER_W

    def chunk_body(c, carry):
        cb = base + c * CHUNK
        pltpu.sync_copy(ids_hbm.at[pl.ds(cb, CHUNK)], ids_c)
        pltpu.sync_copy(pos_hbm.at[pl.ds(cb, CHUNK)], pos_c)
        ct = pltpu.async_copy(tok_tbl.at[ids_c], tok_rows, sem_t)
        cp = pltpu.async_copy(pos_tbl.at[pos_c], pos_rows, sem_p)
        ct.wait()
        cp.wait()

        def add_row(r, carry2):
            def add_vec(j, carry3):
                sl = pl.ds(j * L, L)
                tok_rows[r, sl] = tok_rows[r, sl] + pos_rows[r, sl]
                return carry3
            return lax.fori_loop(0, D // L, add_vec, carry2)

        lax.fori_loop(0, CHUNK, add_row, None)
        pltpu.sync_copy(tok_rows, out_hbm.at[pl.ds(cb, CHUNK)])
        return carry

    lax.fori_loop(0, NCHUNK, chunk_body, None)


@jax.jit
def kernel(input_ids, seq_pos, embed_tokens, seq_pos_embedding):
    ids = input_ids.reshape(N).astype(jnp.int32)
    pos = seq_pos.reshape(N).astype(jnp.int32)
    out = _embed_sum(ids, pos, embed_tokens, seq_pos_embedding)
    return out.reshape(B, S, D)


# speedup vs baseline: 1.9812x; 1.9812x over previous
import functools

import jax
import jax.numpy as jnp
from jax import lax
from jax.experimental import pallas as pl
from jax.experimental.pallas import tpu as pltpu
from jax.experimental.pallas import tpu_sc as plsc

NC, NS, L = 2, 16, 16
NW = NC * NS

B, S, D = 4, 2048, 768
N = B * S
PER_W = N // NW
CHUNK = 32
NCHUNK = PER_W // CHUNK

_mesh = plsc.VectorSubcoreMesh(core_axis_name="c", subcore_axis_name="s")


@functools.partial(
    pl.kernel,
    out_type=jax.ShapeDtypeStruct((N, D), jnp.float32),
    mesh=_mesh,
    scratch_types=[
        pltpu.VMEM((NCHUNK, CHUNK), jnp.int32),
        pltpu.VMEM((NCHUNK, CHUNK), jnp.int32),
        pltpu.VMEM((CHUNK, D), jnp.float32),
        pltpu.VMEM((CHUNK, D), jnp.float32),
        pltpu.VMEM((CHUNK, D), jnp.float32),
        pltpu.VMEM((CHUNK, D), jnp.float32),
        pltpu.SemaphoreType.DMA,
        pltpu.SemaphoreType.DMA,
        pltpu.SemaphoreType.DMA,
        pltpu.SemaphoreType.DMA,
        pltpu.SemaphoreType.DMA,
        pltpu.SemaphoreType.DMA,
    ],
)
def _embed_sum(ids_hbm, pos_hbm, tok_tbl, pos_tbl, out_hbm,
               ids_all, pos_all, tok0, tok1, pr0, pr1,
               st0, st1, sp0, sp1, ss0, ss1):
    wid = lax.axis_index("s") * NC + lax.axis_index("c")
    base = wid * PER_W
    tok_rows = (tok0, tok1)
    pos_rows = (pr0, pr1)
    sem_t = (st0, st1)
    sem_p = (sp0, sp1)
    sem_s = (ss0, ss1)

    pltpu.sync_copy(ids_hbm.at[wid], ids_all)
    pltpu.sync_copy(pos_hbm.at[wid], pos_all)

    def issue(c):
        b = c % 2
        ct = pltpu.async_copy(tok_tbl.at[ids_all.at[c]], tok_rows[b], sem_t[b])
        cp = pltpu.async_copy(pos_tbl.at[pos_all.at[c]], pos_rows[b], sem_p[b])
        return ct, cp

    pending = [None, None]
    stores = [None, None]
    pending[0] = issue(0)
    for c in range(NCHUNK):
        b = c % 2
        if c + 1 < NCHUNK:
            if stores[1 - b] is not None:
                stores[1 - b].wait()
            pending[1 - b] = issue(c + 1)
        ct, cp = pending[b]
        ct.wait()
        cp.wait()

        tr, pr = tok_rows[b], pos_rows[b]

        def add_row(r, carry, tr=tr, pr=pr):
            for j in range(D // L):
                sl = pl.ds(j * L, L)
                tr[r, sl] = tr[r, sl] + pr[r, sl]
            return carry

        lax.fori_loop(0, CHUNK, add_row, None)
        stores[b] = pltpu.async_copy(
            tr, out_hbm.at[pl.ds(base + c * CHUNK, CHUNK)], sem_s[b])
    for st in stores:
        if st is not None:
            st.wait()


@jax.jit
def kernel(input_ids, seq_pos, embed_tokens, seq_pos_embedding):
    ids = input_ids.reshape(NW, NCHUNK, CHUNK).astype(jnp.int32)
    pos = seq_pos.reshape(NW, NCHUNK, CHUNK).astype(jnp.int32)
    out = _embed_sum(ids, pos, embed_tokens, seq_pos_embedding)
    return out.reshape(B, S, D)


# speedup vs baseline: 1.9827x; 1.0008x over previous
import functools

import jax
import jax.numpy as jnp
from jax import lax
from jax.experimental import pallas as pl
from jax.experimental.pallas import tpu as pltpu
from jax.experimental.pallas import tpu_sc as plsc

NC, NS, L = 2, 16, 16
NW = NC * NS

B, S, D = 4, 2048, 768
N = B * S
PER_W = N // NW
CHUNK = 32
NCHUNK = PER_W // CHUNK

_mesh = plsc.VectorSubcoreMesh(core_axis_name="c", subcore_axis_name="s")


@functools.partial(
    pl.kernel,
    out_type=jax.ShapeDtypeStruct((N, D), jnp.float32),
    mesh=_mesh,
    scratch_types=[
        pltpu.VMEM((NCHUNK, CHUNK), jnp.int32),
        pltpu.VMEM((NCHUNK, CHUNK), jnp.int32),
        pltpu.VMEM((CHUNK, D), jnp.float32),
        pltpu.VMEM((CHUNK, D), jnp.float32),
        pltpu.VMEM((CHUNK, D), jnp.float32),
        pltpu.VMEM((CHUNK, D), jnp.float32),
        pltpu.SemaphoreType.DMA,
        pltpu.SemaphoreType.DMA,
        pltpu.SemaphoreType.DMA,
        pltpu.SemaphoreType.DMA,
        pltpu.SemaphoreType.DMA,
        pltpu.SemaphoreType.DMA,
    ],
)
def _embed_sum(ids_hbm, pos_hbm, tok_tbl, pos_tbl, out_hbm,
               ids_all, pos_all, tok0, tok1, pr0, pr1,
               st0, st1, sp0, sp1, ss0, ss1):
    wid = lax.axis_index("s") * NC + lax.axis_index("c")
    base = wid * PER_W
    tok_rows = (tok0, tok1)
    pos_rows = (pr0, pr1)
    sem_t = (st0, st1)
    sem_p = (sp0, sp1)
    sem_s = (ss0, ss1)

    with jax.named_scope("stage_idx"):
        pltpu.sync_copy(ids_hbm.at[wid], ids_all)
        pltpu.sync_copy(pos_hbm.at[wid], pos_all)

    def issue(c):
        b = c % 2
        ct = pltpu.async_copy(tok_tbl.at[ids_all.at[c]], tok_rows[b], sem_t[b])
        cp = pltpu.async_copy(pos_tbl.at[pos_all.at[c]], pos_rows[b], sem_p[b])
        return ct, cp

    pending = [None, None]
    stores = [None, None]
    pending[0] = issue(0)
    for c in range(NCHUNK):
        b = c % 2
        if c + 1 < NCHUNK:
            if stores[1 - b] is not None:
                with jax.named_scope("swait"):
                    stores[1 - b].wait()
            pending[1 - b] = issue(c + 1)
        with jax.named_scope("gwait"):
            ct, cp = pending[b]
            ct.wait()
            cp.wait()

        tr, pr = tok_rows[b], pos_rows[b]

        def add_row(r, carry, tr=tr, pr=pr):
            for j in range(D // L):
                sl = pl.ds(j * L, L)
                tr[r, sl] = tr[r, sl] + pr[r, sl]
            return carry

        with jax.named_scope("add"):
            lax.fori_loop(0, CHUNK, add_row, None)
        stores[b] = pltpu.async_copy(
            tr, out_hbm.at[pl.ds(base + c * CHUNK, CHUNK)], sem_s[b])
    for st in stores:
        if st is not None:
            st.wait()


@jax.jit
def kernel(input_ids, seq_pos, embed_tokens, seq_pos_embedding):
    ids = input_ids.reshape(NW, NCHUNK, CHUNK).astype(jnp.int32)
    pos = seq_pos.reshape(NW, NCHUNK, CHUNK).astype(jnp.int32)
    out = _embed_sum(ids, pos, embed_tokens, seq_pos_embedding)
    return out.reshape(B, S, D)
